# baseline (speedup 1.0000x reference)
import functools

import jax
import jax.numpy as jnp
from jax import lax
from jax.experimental import pallas as pl
from jax.experimental.pallas import tpu as pltpu
from jax.experimental.pallas import tpu_sc as plsc

B, D, V = 128, 1024, 100000
SL = 12544
NSL = (V + SL - 1) // SL
TK = 128
NKP = D // TK
GCH = 256
NGC = SL // GCH

_ROT0 = (13, 15, 26, 6)
_ROT1 = (17, 29, 16, 24)
_TINY = float(jnp.finfo(jnp.float32).tiny)


def _gumbel_chunk(p):
    ks0 = jnp.uint32(0)
    ks1 = jnp.uint32(42)
    ks2 = jnp.uint32(0x1BD11BDA) ^ ks1

    def rnd(x0, x1, r):
        x0 = x0 + x1
        x1 = (x1 << r) | (x1 >> (32 - r))
        x1 = x0 ^ x1
        return x0, x1

    x0 = jnp.zeros(p.shape, jnp.uint32) + ks0
    x1 = p + ks1
    for r in _ROT0:
        x0, x1 = rnd(x0, x1, r)
    x0 = x0 + ks1
    x1 = x1 + ks2 + jnp.uint32(1)
    for r in _ROT1:
        x0, x1 = rnd(x0, x1, r)
    x0 = x0 + ks2
    x1 = x1 + ks0 + jnp.uint32(2)
    for r in _ROT0:
        x0, x1 = rnd(x0, x1, r)
    x0 = x0 + ks0
    x1 = x1 + ks1 + jnp.uint32(3)
    for r in _ROT1:
        x0, x1 = rnd(x0, x1, r)
    x0 = x0 + ks1
    x1 = x1 + ks2 + jnp.uint32(4)
    for r in _ROT0:
        x0, x1 = rnd(x0, x1, r)
    x0 = x0 + ks2
    x1 = x1 + ks0 + jnp.uint32(5)
    bits = x0 ^ x1

    fb = (bits >> 9) | jnp.uint32(0x3F800000)
    f = lax.bitcast_convert_type(fb, jnp.float32) - jnp.float32(1.0)
    tiny = jnp.float32(_TINY)
    u = jnp.maximum(tiny, f * (jnp.float32(1.0) - tiny) + tiny)
    return -jnp.log(-jnp.log(u))


def _tc_body(x_ref, w_ref, b_ref, raw_ref, val_ref, lse_ref,
             m_ref, s_ref, bb_ref, bi_ref):
    ks = pl.program_id(0)
    kk = pl.program_id(1)

    part = jnp.dot(x_ref[...], w_ref[...],
                   preferred_element_type=jnp.float32)

    @pl.when(kk == 0)
    def _():
        raw_ref[...] = part + b_ref[...]

    @pl.when(kk > 0)
    def _():
        raw_ref[...] = raw_ref[...] + part

    @pl.when(jnp.logical_and(ks == 0, kk == 0))
    def _():
        neg = jnp.float32(-jnp.inf)
        m_ref[...] = jnp.full((B, 1), neg, jnp.float32)
        s_ref[...] = jnp.zeros((B, 1), jnp.float32)
        bb_ref[...] = jnp.full((B, 1), neg, jnp.float32)
        bi_ref[...] = jnp.zeros((B, 1), jnp.int32)

    @pl.when(kk == NKP - 1)
    def _():
        neg = jnp.float32(-jnp.inf)
        m_run = m_ref[...]
        s_run = s_ref[...]
        bb_run = bb_ref[...]
        bi_run = bi_ref[...]
        rowu = lax.broadcasted_iota(jnp.uint32, (B, GCH), 0)
        for c in range(NGC):
            base = ks * SL + c * GCH
            raw_c = raw_ref[:, pl.ds(c * GCH, GCH)]
            cols = base + lax.broadcasted_iota(jnp.int32, (B, GCH), 1)
            valid = cols < V
            p = (rowu * jnp.uint32(V) +
                 lax.broadcasted_iota(jnp.uint32, (B, GCH), 1) +
                 base.astype(jnp.uint32))
            z = jnp.where(valid, raw_c + _gumbel_chunk(p), neg)
            rm = jnp.where(valid, raw_c, neg)

            tmax = jnp.max(rm, axis=1, keepdims=True)
            zmax = jnp.max(z, axis=1, keepdims=True)
            zidx = jnp.min(jnp.where(z == zmax, cols, V),
                           axis=1, keepdims=True)

            m_new = jnp.maximum(m_run, tmax)
            e = jnp.where(valid, jnp.exp(raw_c - m_new), 0.0)
            s_run = s_run * jnp.exp(m_run - m_new) + jnp.sum(
                e, axis=1, keepdims=True)
            m_run = m_new

            upd = zmax > bb_run
            bi_run = jnp.where(upd, zidx, bi_run)
            bb_run = jnp.where(upd, zmax, bb_run)
        m_ref[...] = m_run
        s_ref[...] = s_run
        bb_ref[...] = bb_run
        bi_ref[...] = bi_run

        @pl.when(ks == NSL - 1)
        def _():
            val_ref[...] = bi_run
            lse_ref[...] = m_run + jnp.log(s_run)


def _tc_call(x, W, b2):
    return pl.pallas_call(
        _tc_body,
        grid=(NSL, NKP),
        in_specs=[
            pl.BlockSpec((B, TK), lambda ks, kk: (0, kk)),
            pl.BlockSpec((TK, SL), lambda ks, kk: (kk, ks)),
            pl.BlockSpec((1, SL), lambda ks, kk: (0, ks)),
        ],
        out_specs=[
            pl.BlockSpec((B, SL), lambda ks, kk: (0, ks)),
            pl.BlockSpec((B, 1), lambda ks, kk: (0, 0)),
            pl.BlockSpec((B, 1), lambda ks, kk: (0, 0)),
        ],
        out_shape=[
            jax.ShapeDtypeStruct((B, V), jnp.float32),
            jax.ShapeDtypeStruct((B, 1), jnp.int32),
            jax.ShapeDtypeStruct((B, 1), jnp.float32),
        ],
        scratch_shapes=[
            pltpu.VMEM((B, 1), jnp.float32),
            pltpu.VMEM((B, 1), jnp.float32),
            pltpu.VMEM((B, 1), jnp.float32),
            pltpu.VMEM((B, 1), jnp.int32),
        ],
        compiler_params=pltpu.CompilerParams(
            dimension_semantics=("arbitrary", "arbitrary")),
    )(x, W, b2)


NC, NS = 2, 16
NW = NC * NS
NT = 8
LPT = B // NT
NE = B * LPT


def _sc_gather_body(fidx_hbm, rawflat_hbm, lserep_hbm, out_hbm,
                    idx_v, gath_v, lse_v, out_v, sem):
    wid = lax.axis_index("s") * NC + lax.axis_index("c")

    @pl.when(wid < NT)
    def _():
        pltpu.sync_copy(fidx_hbm.at[wid], idx_v)
        pltpu.sync_copy(lserep_hbm, lse_v)
        copies = [
            pltpu.async_copy(rawflat_hbm.at[idx_v.at[r]], gath_v.at[r], sem)
            for r in range(16)
        ]
        for cp in copies:
            cp.wait()

        acc_t = jnp.zeros((16,), jnp.float32)
        acc_p = jnp.zeros((16,), jnp.float32)
        for r in range(16):
            for cc in range(8):
                c = gath_v[r, pl.ds(cc * 16, 16)]
                l = lse_v[pl.ds((r * 8 + cc) * 16, 16)]
                t = c - l
                acc_t = acc_t + t
                acc_p = acc_p + jnp.exp(t)
        inv_b = jnp.float32(1.0 / B)
        out_v[pl.ds(0, 16)] = acc_p * inv_b
        out_v[pl.ds(16, 16)] = acc_t * inv_b
        pltpu.sync_copy(out_v, out_hbm.at[wid])


@functools.cache
def _sc_gather_call():
    mesh = plsc.VectorSubcoreMesh(core_axis_name="c", subcore_axis_name="s")
    return pl.kernel(
        _sc_gather_body,
        out_type=jax.ShapeDtypeStruct((NT, 2 * LPT), jnp.float32),
        mesh=mesh,
        scratch_types=[
            pltpu.VMEM((16, B), jnp.int32),
            pltpu.VMEM((16, B), jnp.float32),
            pltpu.VMEM((NE,), jnp.float32),
            pltpu.VMEM((2 * LPT,), jnp.float32),
            pltpu.SemaphoreType.DMA,
        ],
    )


def kernel(x, W, b):
    raw, val2, lse2 = _tc_call(x, W, b.reshape(1, V))
    value = val2[:, 0]
    ii = jnp.arange(B, dtype=jnp.int32) * V
    fidx = (ii[None, :, None] +
            value.reshape(NT, 1, LPT)).reshape(NT, 16, B)
    lse = lse2[:, 0]
    lse_rep = jnp.repeat(lse, LPT)
    out = _sc_gather_call()(fidx, raw.reshape(B * V), lse_rep)
    sampled = out[:, 0:LPT].reshape(B)
    log_sampled = out[:, LPT:2 * LPT].reshape(B)
    return raw, value, sampled, log_sampled


# speedup vs baseline: 1.4329x; 1.4329x over previous
import functools

import jax
import jax.numpy as jnp
from jax import lax
from jax.experimental import pallas as pl
from jax.experimental.pallas import tpu as pltpu
from jax.experimental.pallas import tpu_sc as plsc

B, D, V = 128, 1024, 100000
TV = 2048
NG = (V + TV - 1) // TV
GCH = 256
NGC = TV // GCH

_ROT0 = (13, 15, 26, 6)
_ROT1 = (17, 29, 16, 24)
_TINY = float(jnp.finfo(jnp.float32).tiny)


def _gumbel_chunk(p):
    ks0 = jnp.uint32(0)
    ks1 = jnp.uint32(42)
    ks2 = jnp.uint32(0x1BD11BDA) ^ ks1

    def rnd(x0, x1, r):
        x0 = x0 + x1
        x1 = (x1 << r) | (x1 >> (32 - r))
        x1 = x0 ^ x1
        return x0, x1

    x0 = jnp.zeros(p.shape, jnp.uint32) + ks0
    x1 = p + ks1
    for r in _ROT0:
        x0, x1 = rnd(x0, x1, r)
    x0 = x0 + ks1
    x1 = x1 + ks2 + jnp.uint32(1)
    for r in _ROT1:
        x0, x1 = rnd(x0, x1, r)
    x0 = x0 + ks2
    x1 = x1 + ks0 + jnp.uint32(2)
    for r in _ROT0:
        x0, x1 = rnd(x0, x1, r)
    x0 = x0 + ks0
    x1 = x1 + ks1 + jnp.uint32(3)
    for r in _ROT1:
        x0, x1 = rnd(x0, x1, r)
    x0 = x0 + ks1
    x1 = x1 + ks2 + jnp.uint32(4)
    for r in _ROT0:
        x0, x1 = rnd(x0, x1, r)
    x0 = x0 + ks2
    x1 = x1 + ks0 + jnp.uint32(5)
    bits = x0 ^ x1

    fb = (bits >> 9) | jnp.uint32(0x3F800000)
    f = lax.bitcast_convert_type(fb, jnp.float32) - jnp.float32(1.0)
    tiny = jnp.float32(_TINY)
    u = jnp.maximum(tiny, f * (jnp.float32(1.0) - tiny) + tiny)
    return -jnp.log(-jnp.log(u))


def _tc_body(x_ref, w_ref, b_ref, raw_ref, val_ref, lse_ref,
             m_ref, s_ref, bb_ref, bi_ref):
    k = pl.program_id(0)

    raw_t = jnp.dot(x_ref[...], w_ref[...],
                    preferred_element_type=jnp.float32) + b_ref[...]
    raw_ref[...] = raw_t

    @pl.when(k == 0)
    def _():
        neg = jnp.float32(-jnp.inf)
        m_ref[...] = jnp.full((B, 1), neg, jnp.float32)
        s_ref[...] = jnp.zeros((B, 1), jnp.float32)
        bb_ref[...] = jnp.full((B, 1), neg, jnp.float32)
        bi_ref[...] = jnp.zeros((B, 1), jnp.int32)

    neg = jnp.float32(-jnp.inf)
    m_run = m_ref[...]
    s_run = s_ref[...]
    bb_run = bb_ref[...]
    bi_run = bi_ref[...]
    rowu = lax.broadcasted_iota(jnp.uint32, (B, GCH), 0)
    coli = lax.broadcasted_iota(jnp.int32, (B, GCH), 1)
    colu = lax.broadcasted_iota(jnp.uint32, (B, GCH), 1)
    for c in range(NGC):
        base = k * TV + c * GCH
        raw_c = raw_t[:, c * GCH:(c + 1) * GCH]
        cols = base + coli
        valid = cols < V
        p = rowu * jnp.uint32(V) + colu + base.astype(jnp.uint32)
        z = jnp.where(valid, raw_c + _gumbel_chunk(p), neg)
        rm = jnp.where(valid, raw_c, neg)

        tmax = jnp.max(rm, axis=1, keepdims=True)
        zmax = jnp.max(z, axis=1, keepdims=True)
        zidx = jnp.min(jnp.where(z == zmax, cols, V),
                       axis=1, keepdims=True)

        m_new = jnp.maximum(m_run, tmax)
        e = jnp.where(valid, jnp.exp(raw_c - m_new), 0.0)
        s_run = s_run * jnp.exp(m_run - m_new) + jnp.sum(
            e, axis=1, keepdims=True)
        m_run = m_new

        upd = zmax > bb_run
        bi_run = jnp.where(upd, zidx, bi_run)
        bb_run = jnp.where(upd, zmax, bb_run)
    m_ref[...] = m_run
    s_ref[...] = s_run
    bb_ref[...] = bb_run
    bi_ref[...] = bi_run

    @pl.when(k == NG - 1)
    def _():
        val_ref[...] = bi_run
        lse_ref[...] = m_run + jnp.log(s_run)


def _tc_call(x, W, b2):
    return pl.pallas_call(
        _tc_body,
        grid=(NG,),
        in_specs=[
            pl.BlockSpec((B, D), lambda k: (0, 0)),
            pl.BlockSpec((D, TV), lambda k: (0, k)),
            pl.BlockSpec((1, TV), lambda k: (0, k)),
        ],
        out_specs=[
            pl.BlockSpec((B, TV), lambda k: (0, k)),
            pl.BlockSpec((B, 1), lambda k: (0, 0)),
            pl.BlockSpec((B, 1), lambda k: (0, 0)),
        ],
        out_shape=[
            jax.ShapeDtypeStruct((B, V), jnp.float32),
            jax.ShapeDtypeStruct((B, 1), jnp.int32),
            jax.ShapeDtypeStruct((B, 1), jnp.float32),
        ],
        scratch_shapes=[
            pltpu.VMEM((B, 1), jnp.float32),
            pltpu.VMEM((B, 1), jnp.float32),
            pltpu.VMEM((B, 1), jnp.float32),
            pltpu.VMEM((B, 1), jnp.int32),
        ],
        compiler_params=pltpu.CompilerParams(
            dimension_semantics=("arbitrary",)),
    )(x, W, b2)


NC, NS = 2, 16
NW = NC * NS
NT = 8
LPT = B // NT
NE = B * LPT


def _sc_gather_body(fidx_hbm, rawflat_hbm, lserep_hbm, out_hbm,
                    idx_v, gath_v, lse_v, out_v, sem):
    wid = lax.axis_index("s") * NC + lax.axis_index("c")

    @pl.when(wid < NT)
    def _():
        pltpu.sync_copy(fidx_hbm.at[wid], idx_v)
        pltpu.sync_copy(lserep_hbm, lse_v)
        copies = [
            pltpu.async_copy(rawflat_hbm.at[idx_v.at[r]], gath_v.at[r], sem)
            for r in range(16)
        ]
        for cp in copies:
            cp.wait()

        acc_t = jnp.zeros((16,), jnp.float32)
        acc_p = jnp.zeros((16,), jnp.float32)
        for r in range(16):
            for cc in range(8):
                c = gath_v[r, pl.ds(cc * 16, 16)]
                l = lse_v[pl.ds((r * 8 + cc) * 16, 16)]
                t = c - l
                acc_t = acc_t + t
                acc_p = acc_p + jnp.exp(t)
        inv_b = jnp.float32(1.0 / B)
        out_v[pl.ds(0, 16)] = acc_p * inv_b
        out_v[pl.ds(16, 16)] = acc_t * inv_b
        pltpu.sync_copy(out_v, out_hbm.at[wid])


@functools.cache
def _sc_gather_call():
    mesh = plsc.VectorSubcoreMesh(core_axis_name="c", subcore_axis_name="s")
    return pl.kernel(
        _sc_gather_body,
        out_type=jax.ShapeDtypeStruct((NT, 2 * LPT), jnp.float32),
        mesh=mesh,
        scratch_types=[
            pltpu.VMEM((16, B), jnp.int32),
            pltpu.VMEM((16, B), jnp.float32),
            pltpu.VMEM((NE,), jnp.float32),
            pltpu.VMEM((2 * LPT,), jnp.float32),
            pltpu.SemaphoreType.DMA,
        ],
    )


def kernel(x, W, b):
    raw, val2, lse2 = _tc_call(x, W, b.reshape(1, V))
    value = val2[:, 0]
    ii = jnp.arange(B, dtype=jnp.int32) * V
    fidx = (ii[None, :, None] +
            value.reshape(NT, 1, LPT)).reshape(NT, 16, B)
    lse = lse2[:, 0]
    lse_rep = jnp.repeat(lse, LPT)
    out = _sc_gather_call()(fidx, raw.reshape(B * V), lse_rep)
    sampled = out[:, 0:LPT].reshape(B)
    log_sampled = out[:, LPT:2 * LPT].reshape(B)
    return raw, value, sampled, log_sampled


# speedup vs baseline: 1.5058x; 1.0508x over previous
import functools

import jax
import jax.numpy as jnp
from jax import lax
from jax.experimental import pallas as pl
from jax.experimental.pallas import tpu as pltpu
from jax.experimental.pallas import tpu_sc as plsc

B, D, V = 128, 1024, 100000
TV = 2048
NG = (V + TV - 1) // TV
GCH = 256
NGC = TV // GCH

_ROT0 = (13, 15, 26, 6)
_ROT1 = (17, 29, 16, 24)
_TINY = float(jnp.finfo(jnp.float32).tiny)


def _gumbel_chunk(p):
    ks0 = jnp.uint32(0)
    ks1 = jnp.uint32(42)
    ks2 = jnp.uint32(0x1BD11BDA) ^ ks1

    def rnd(x0, x1, r):
        x0 = x0 + x1
        x1 = (x1 << r) | (x1 >> (32 - r))
        x1 = x0 ^ x1
        return x0, x1

    x0 = jnp.zeros(p.shape, jnp.uint32) + ks0
    x1 = p + ks1
    for r in _ROT0:
        x0, x1 = rnd(x0, x1, r)
    x0 = x0 + ks1
    x1 = x1 + ks2 + jnp.uint32(1)
    for r in _ROT1:
        x0, x1 = rnd(x0, x1, r)
    x0 = x0 + ks2
    x1 = x1 + ks0 + jnp.uint32(2)
    for r in _ROT0:
        x0, x1 = rnd(x0, x1, r)
    x0 = x0 + ks0
    x1 = x1 + ks1 + jnp.uint32(3)
    for r in _ROT1:
        x0, x1 = rnd(x0, x1, r)
    x0 = x0 + ks1
    x1 = x1 + ks2 + jnp.uint32(4)
    for r in _ROT0:
        x0, x1 = rnd(x0, x1, r)
    x0 = x0 + ks2
    x1 = x1 + ks0 + jnp.uint32(5)
    bits = x0 ^ x1

    fb = (bits >> 9) | jnp.uint32(0x3F800000)
    f = lax.bitcast_convert_type(fb, jnp.float32) - jnp.float32(1.0)
    tiny = jnp.float32(_TINY)
    u = jnp.maximum(tiny, f * (jnp.float32(1.0) - tiny) + tiny)
    return -jnp.log(-jnp.log(u))


def _tc_body(x_ref, w_ref, b_ref, raw_ref, val_ref, lse_ref,
             m_ref, s_ref, bb_ref, bi_ref):
    k = pl.program_id(0)

    raw_t = jnp.dot(x_ref[...], w_ref[...],
                    preferred_element_type=jnp.float32) + b_ref[...]
    raw_ref[...] = raw_t

    @pl.when(k == 0)
    def _():
        neg = jnp.float32(-jnp.inf)
        m_ref[...] = jnp.full((B, 1), neg, jnp.float32)
        s_ref[...] = jnp.zeros((B, 1), jnp.float32)
        bb_ref[...] = jnp.full((B, 1), neg, jnp.float32)
        bi_ref[...] = jnp.zeros((B, 1), jnp.int32)

    neg = jnp.float32(-jnp.inf)
    rowu = lax.broadcasted_iota(jnp.uint32, (B, TV), 0)
    cols = k * TV + lax.broadcasted_iota(jnp.int32, (B, TV), 1)
    valid = cols < V
    p = (rowu * jnp.uint32(V) +
         lax.broadcasted_iota(jnp.uint32, (B, TV), 1) +
         (k * TV).astype(jnp.uint32))
    rm = jnp.where(valid, raw_t, neg)
    z = jnp.where(valid, raw_t + _gumbel_chunk(p), neg)

    tmax = jnp.max(rm, axis=1, keepdims=True)
    zmax = jnp.max(z, axis=1, keepdims=True)
    zidx = jnp.min(jnp.where(z == zmax, cols, V), axis=1, keepdims=True)

    m_old = m_ref[...]
    m_new = jnp.maximum(m_old, tmax)
    e = jnp.where(valid, jnp.exp(raw_t - m_new), 0.0)
    s_ref[...] = s_ref[...] * jnp.exp(m_old - m_new) + jnp.sum(
        e, axis=1, keepdims=True)
    m_ref[...] = m_new

    bb_old = bb_ref[...]
    upd = zmax > bb_old
    bi_run = jnp.where(upd, zidx, bi_ref[...])
    bb_ref[...] = jnp.where(upd, zmax, bb_old)
    bi_ref[...] = bi_run
    m_run = m_new
    s_run = s_ref[...]

    @pl.when(k == NG - 1)
    def _():
        val_ref[...] = bi_run
        lse_ref[...] = m_run + jnp.log(s_run)


def _tc_call(x, W, b2):
    return pl.pallas_call(
        _tc_body,
        grid=(NG,),
        in_specs=[
            pl.BlockSpec((B, D), lambda k: (0, 0)),
            pl.BlockSpec((D, TV), lambda k: (0, k)),
            pl.BlockSpec((1, TV), lambda k: (0, k)),
        ],
        out_specs=[
            pl.BlockSpec((B, TV), lambda k: (0, k)),
            pl.BlockSpec((B, 1), lambda k: (0, 0)),
            pl.BlockSpec((B, 1), lambda k: (0, 0)),
        ],
        out_shape=[
            jax.ShapeDtypeStruct((B, V), jnp.float32),
            jax.ShapeDtypeStruct((B, 1), jnp.int32),
            jax.ShapeDtypeStruct((B, 1), jnp.float32),
        ],
        scratch_shapes=[
            pltpu.VMEM((B, 1), jnp.float32),
            pltpu.VMEM((B, 1), jnp.float32),
            pltpu.VMEM((B, 1), jnp.float32),
            pltpu.VMEM((B, 1), jnp.int32),
        ],
        compiler_params=pltpu.CompilerParams(
            dimension_semantics=("arbitrary",)),
    )(x, W, b2)


NC, NS = 2, 16
NW = NC * NS
NT = 8
LPT = B // NT
NE = B * LPT


def _sc_gather_body(fidx_hbm, rawflat_hbm, lserep_hbm, out_hbm,
                    idx_v, gath_v, lse_v, out_v, sem):
    wid = lax.axis_index("s") * NC + lax.axis_index("c")

    @pl.when(wid < NT)
    def _():
        pltpu.sync_copy(fidx_hbm.at[wid], idx_v)
        pltpu.sync_copy(lserep_hbm, lse_v)
        copies = [
            pltpu.async_copy(rawflat_hbm.at[idx_v.at[r]], gath_v.at[r], sem)
            for r in range(16)
        ]
        for cp in copies:
            cp.wait()

        acc_t = jnp.zeros((16,), jnp.float32)
        acc_p = jnp.zeros((16,), jnp.float32)
        for r in range(16):
            for cc in range(8):
                c = gath_v[r, pl.ds(cc * 16, 16)]
                l = lse_v[pl.ds((r * 8 + cc) * 16, 16)]
                t = c - l
                acc_t = acc_t + t
                acc_p = acc_p + jnp.exp(t)
        inv_b = jnp.float32(1.0 / B)
        out_v[pl.ds(0, 16)] = acc_p * inv_b
        out_v[pl.ds(16, 16)] = acc_t * inv_b
        pltpu.sync_copy(out_v, out_hbm.at[wid])


@functools.cache
def _sc_gather_call():
    mesh = plsc.VectorSubcoreMesh(core_axis_name="c", subcore_axis_name="s")
    return pl.kernel(
        _sc_gather_body,
        out_type=jax.ShapeDtypeStruct((NT, 2 * LPT), jnp.float32),
        mesh=mesh,
        scratch_types=[
            pltpu.VMEM((16, B), jnp.int32),
            pltpu.VMEM((16, B), jnp.float32),
            pltpu.VMEM((NE,), jnp.float32),
            pltpu.VMEM((2 * LPT,), jnp.float32),
            pltpu.SemaphoreType.DMA,
        ],
    )


def kernel(x, W, b):
    raw, val2, lse2 = _tc_call(x, W, b.reshape(1, V))
    value = val2[:, 0]
    ii = jnp.arange(B, dtype=jnp.int32) * V
    fidx = (ii[None, :, None] +
            value.reshape(NT, 1, LPT)).reshape(NT, 16, B)
    lse = lse2[:, 0]
    lse_rep = jnp.repeat(lse, LPT)
    out = _sc_gather_call()(fidx, raw.reshape(B * V), lse_rep)
    sampled = out[:, 0:LPT].reshape(B)
    log_sampled = out[:, LPT:2 * LPT].reshape(B)
    return raw, value, sampled, log_sampled
